# baseline (speedup 1.0000x reference)
import functools

import jax
import jax.numpy as jnp
from jax import lax
from jax.experimental import pallas as pl
from jax.experimental.pallas import tpu as pltpu
from jax.experimental.pallas import tpu_sc as plsc

_N = 10000
_F = 128
_T = 12
_NC = 2
_NS = 16
_NW = _NC * _NS
_NP = 10112
_RPT = _NP // _NS
_CHUNK = 128
_GPW = 160000 // _NW
_RPW = 160000 // _NW
_EPW = 10240
_NCH = _EPW // _CHUNK

_SCALE_FROM = _GPW // _CHUNK
_WSEC = (_NCH - _SCALE_FROM) * _CHUNK


def _prep_edges(edge_index, regs):
    gsrc = edge_index[0].reshape(_NW, _GPW)
    gdst = edge_index[1].reshape(_NW, _GPW)
    gw = jnp.ones((_NW, _GPW), jnp.float32)
    rsrc = jnp.concatenate([ei[0] for ei, _ in regs]).reshape(_NW, _RPW)
    rdst = jnp.concatenate([ei[1] for ei, _ in regs]).reshape(_NW, _RPW)
    rw = jnp.concatenate([ew for _, ew in regs]).reshape(_NW, _RPW)
    pad = _EPW - _GPW - _RPW
    zi = jnp.zeros((_NW, pad), jnp.int32)
    src = jnp.concatenate([gsrc, rsrc, zi], axis=1)
    dst = jnp.concatenate([gdst, rdst, zi], axis=1)
    w = jnp.concatenate([gw, rw, jnp.zeros((_NW, pad), jnp.float32)], axis=1)
    w_sec = w[:, _SCALE_FROM * _CHUNK:]
    r3 = lambda a: a.reshape(_NW, _NCH, _CHUNK)
    return r3(src), r3(dst), w_sec


def _sc_segsum(xT, ztile, srcv_h, dst16, w_sec):
    @functools.partial(
        pl.kernel,
        out_type=jax.ShapeDtypeStruct((_NC, _T, _NP, _F), jnp.float32),
        mesh=plsc.VectorSubcoreMesh(core_axis_name="c", subcore_axis_name="s"),
        scratch_types=[
            pltpu.VMEM((_NCH, _CHUNK), jnp.int32),
            pltpu.VMEM((_NCH, _CHUNK), jnp.int32),
            pltpu.VMEM((_WSEC,), jnp.float32),
            pltpu.VMEM((_CHUNK, _F), jnp.float32),
            pltpu.VMEM_SHARED((_NP, _F), jnp.float32),
            pltpu.SemaphoreType.DMA,
        ],
    )
    def body(x_hbm, z_hbm, src_hbm, dst_hbm, w_hbm, out_hbm,
             srcv, dstv, ws, rows, accum, sem):
        c = lax.axis_index("c")
        s = lax.axis_index("s")
        wid = c * _NS + s
        pltpu.sync_copy(src_hbm.at[wid], srcv)
        pltpu.sync_copy(dst_hbm.at[wid], dstv)
        pltpu.sync_copy(w_hbm.at[wid], ws)

        def per_t(t, carry):
            @pl.when(c == 0)
            def _():
                pltpu.sync_copy(x_hbm.at[pl.ds(t * _N + s * _RPT, _RPT)],
                                accum.at[pl.ds(s * _RPT, _RPT)])

            @pl.when(c != 0)
            def _():
                pltpu.sync_copy(z_hbm, accum.at[pl.ds(s * _RPT, _RPT)])

            plsc.subcore_barrier()

            def per_chunk(j, cc):
                pltpu.async_copy(x_hbm.at[srcv.at[j]], rows, sem).wait()

                @pl.when(j >= _SCALE_FROM)
                def _():
                    def per_group(g, rc):
                        wvec = ws[pl.ds((j - _SCALE_FROM) * _CHUNK + g * 16,
                                        16)]
                        for rr in range(16):
                            wb = wvec[rr]
                            r = g * 16 + rr
                            for f in range(_F // 16):
                                sl = pl.ds(f * 16, 16)
                                rows[r, sl] = rows[r, sl] * wb
                        return rc

                    lax.fori_loop(0, _CHUNK // 16, per_group, 0)

                pltpu.sync_copy(rows, accum.at[dstv.at[j]], add=True)
                return cc

            lax.fori_loop(0, _NCH, per_chunk, 0)

            def adv_body(j, cc):
                for k in range(_CHUNK // 16):
                    sl = pl.ds(k * 16, 16)
                    srcv[j, sl] = srcv[j, sl] + _N
                return cc

            lax.fori_loop(0, _NCH, adv_body, 0)
            plsc.subcore_barrier()
            pltpu.sync_copy(accum.at[pl.ds(s * _RPT, _RPT)],
                            out_hbm.at[c, t, pl.ds(s * _RPT, _RPT)])
            return carry

        lax.fori_loop(0, _T, per_t, 0)

    return body(xT, ztile, srcv_h, dst16, w_sec)


_BN = 1000


def _tc_body(S_ref, Wz_ref, bz_ref, Wr_ref, br_ref, Wh_ref, bh_ref,
             Lz_ref, blz_ref, Lr_ref, blr_ref, Lh_ref, blh_ref,
             att_ref, W1_ref, b1_ref, W2_ref, b2_ref, y_ref, hacc_ref):
    f32 = jnp.float32
    Lzv, Lrv, Lhv = Lz_ref[...], Lr_ref[...], Lh_ref[...]
    Lzt, Lzb = Lzv[:256], Lzv[256:]
    Lrt, Lrb = Lrv[:256], Lrv[256:]
    Lht, Lhb = Lhv[:256], Lhv[256:]
    M8 = jnp.concatenate([
        jnp.dot(Wz_ref[...], Lzt, preferred_element_type=f32),
        jnp.dot(Wr_ref[...], Lrt, preferred_element_type=f32),
        jnp.dot(Wh_ref[...], Lht, preferred_element_type=f32)], axis=1) * 0.125
    cvec = jnp.concatenate([
        jnp.dot(bz_ref[...], Lzt, preferred_element_type=f32) + blz_ref[...],
        jnp.dot(br_ref[...], Lrt, preferred_element_type=f32) + blr_ref[...],
        jnp.dot(bh_ref[...], Lht, preferred_element_type=f32) + blh_ref[...]],
        axis=1)
    Lzr = jnp.concatenate([Lzb, Lrb], axis=1)
    probs = jax.nn.softmax(att_ref[...], axis=-1)

    def step(t, carry):
        H, Hacc = carry
        A = S_ref[pl.ds(t, 1)][0] + S_ref[pl.ds(t + _T, 1)][0]
        P = jnp.dot(A, M8, preferred_element_type=f32) + cvec
        ZR = P[:, :512] + jnp.dot(H, Lzr, preferred_element_type=f32)
        Z = jax.nn.sigmoid(ZR[:, :256])
        R = jax.nn.sigmoid(ZR[:, 256:])
        Htil = jnp.tanh(P[:, 512:]
                        + jnp.dot(H * R, Lhb, preferred_element_type=f32))
        Hn = Z * H + (1.0 - Z) * Htil
        mask = lax.broadcasted_iota(jnp.int32, (1, _T), 1) == t
        pt = jnp.sum(jnp.where(mask, probs, 0.0))
        return Hn, Hacc + pt * Hn

    H0 = jnp.zeros((_BN, 256), f32)
    _, Hacc = lax.fori_loop(0, _T, step, (H0, H0))
    h = jnp.maximum(Hacc, 0.0)
    h = jnp.maximum(
        jnp.dot(h, W1_ref[...], preferred_element_type=f32) + b1_ref[...], 0.0)
    y = jnp.dot(h, W2_ref[...], preferred_element_type=f32) + b2_ref[...]
    y_ref[...] = y
    hacc_ref[...] = Hacc


def _tc_gru(S2, Wz, bz, Wr, br, Wh, bh, Lz, blz, Lr, blr, Lh, blh,
            att, W1, b1, W2, b2):
    full = lambda shape: pl.BlockSpec(shape, lambda i: (0,) * len(shape))
    return pl.pallas_call(
        _tc_body,
        grid=(_N // _BN,),
        in_specs=[
            pl.BlockSpec((_NC * _T, _BN, _F), lambda i: (0, i, 0)),
            full((_F, 256)), full((1, 256)),
            full((_F, 256)), full((1, 256)),
            full((_F, 256)), full((1, 256)),
            full((512, 256)), full((1, 256)),
            full((512, 256)), full((1, 256)),
            full((512, 256)), full((1, 256)),
            full((1, _T)),
            full((256, 128)), full((1, 128)),
            full((128, 1)), full((1, 1)),
        ],
        out_specs=[
            pl.BlockSpec((_BN, 1), lambda i: (i, 0)),
            pl.BlockSpec((_BN, 256), lambda i: (i, 0)),
        ],
        out_shape=[
            jax.ShapeDtypeStruct((_N, 1), jnp.float32),
            jax.ShapeDtypeStruct((_N, 256), jnp.float32),
        ],
        compiler_params=pltpu.CompilerParams(
            dimension_semantics=("arbitrary",)),
    )(S2, Wz, bz, Wr, br, Wh, bh, Lz, blz, Lr, blr, Lh, blh,
      att, W1, b1, W2, b2)


def kernel(x, edge_index, IAedge_index, KSedge_index, KYedge_index,
           OHedge_index, WIedge_index, IAedge_attr, KSedge_attr, KYedge_attr,
           OHedge_attr, WIedge_attr, Wz, bz, Wr, br, Wh, bh, Lz, blz, Lr, blr,
           Lh, blh, att, W1, b1, W2, b2):
    xT = jnp.transpose(x, (2, 0, 1)).reshape(_T * _N, _F)
    xT = jnp.concatenate(
        [xT, jnp.zeros((_NP - _N, _F), jnp.float32)], axis=0)
    regs = [(IAedge_index, IAedge_attr), (KSedge_index, KSedge_attr),
            (KYedge_index, KYedge_attr), (OHedge_index, OHedge_attr),
            (WIedge_index, WIedge_attr)]
    srcv_h, dst16, w_sec = _prep_edges(edge_index, regs)
    ztile = jnp.zeros((_RPT, _F), jnp.float32)
    S = _sc_segsum(xT, ztile, srcv_h, dst16, w_sec)
    S2 = S.reshape(_NC * _T, _NP, _F)
    r2 = lambda v: v.reshape(1, -1)
    y, hacc = _tc_gru(S2, Wz, r2(bz), Wr, r2(br), Wh, r2(bh),
                      Lz, r2(blz), Lr, r2(blr), Lh, r2(blh),
                      r2(att), W1, r2(b1), W2, r2(b2))
    return (y, hacc)


# speedup vs baseline: 1.4501x; 1.4501x over previous
import functools

import jax
import jax.numpy as jnp
from jax import lax
from jax.experimental import pallas as pl
from jax.experimental.pallas import tpu as pltpu
from jax.experimental.pallas import tpu_sc as plsc

_N = 10000
_F = 128
_T = 12
_NC = 2
_NS = 16
_NW = _NC * _NS
_NP = 10240
_RPT = _NP // _NS
_CHUNK = 128
_GPW = 160000 // _NW
_RPW = 160000 // _NW
_EPW = 10112
_NCH = _EPW // _CHUNK

_SCALE_FROM = _GPW // _CHUNK


def _prep_edges(edge_index, regs):
    gsrc = edge_index[0].reshape(_NW, _GPW)
    gdst = edge_index[1].reshape(_NW, _GPW)
    gw = jnp.ones((_NW, _GPW), jnp.float32)
    rsrc = jnp.concatenate([ei[0] for ei, _ in regs]).reshape(_NW, _RPW)
    rdst = jnp.concatenate([ei[1] for ei, _ in regs]).reshape(_NW, _RPW)
    rw = jnp.concatenate([ew for _, ew in regs]).reshape(_NW, _RPW)
    pad = _EPW - _GPW - _RPW
    zi = jnp.zeros((_NW, pad), jnp.int32)
    src = jnp.concatenate([gsrc, rsrc, zi], axis=1)
    dst = jnp.concatenate([gdst, rdst, zi], axis=1)
    w = jnp.concatenate([gw, rw, jnp.zeros((_NW, pad), jnp.float32)], axis=1)
    r3 = lambda a: a.reshape(_NW, _NCH, _CHUNK)
    return r3(src), r3(dst), w


def _sc_segsum(xT, ztile, srcv_h, dst16, w_sec):
    @functools.partial(
        pl.kernel,
        out_type=jax.ShapeDtypeStruct((_NC, _T, _NP, _F), jnp.float32),
        mesh=plsc.VectorSubcoreMesh(core_axis_name="c", subcore_axis_name="s"),
        scratch_types=[
            pltpu.VMEM((_NCH, _CHUNK), jnp.int32),
            pltpu.VMEM((_NCH, _CHUNK), jnp.int32),
            pltpu.VMEM((_EPW,), jnp.float32),
            pltpu.VMEM((_CHUNK, _F), jnp.float32),
            pltpu.VMEM_SHARED((_NP, _F), jnp.float32),
            pltpu.SemaphoreType.DMA,
        ],
    )
    def body(x_hbm, z_hbm, src_hbm, dst_hbm, w_hbm, out_hbm,
             srcv, dstv, ws, rows, accum, sem):
        c = lax.axis_index("c")
        s = lax.axis_index("s")
        wid = c * _NS + s
        pltpu.sync_copy(src_hbm.at[wid], srcv)
        pltpu.sync_copy(dst_hbm.at[wid], dstv)
        pltpu.sync_copy(w_hbm.at[wid], ws)

        def per_t(t, carry):
            @pl.when(c == 0)
            def _():
                pltpu.sync_copy(x_hbm.at[pl.ds(t * _N + s * _RPT, _RPT)],
                                accum.at[pl.ds(s * _RPT, _RPT)])

            @pl.when(c != 0)
            def _():
                pltpu.sync_copy(z_hbm, accum.at[pl.ds(s * _RPT, _RPT)])

            plsc.subcore_barrier()

            def per_chunk(j, cc):
                pltpu.async_copy(x_hbm.at[srcv.at[j]], rows, sem).wait()

                @pl.when(j >= _SCALE_FROM)
                def _():
                    def per_group(g, rc):
                        wvec = ws[pl.ds(j * _CHUNK + g * 16, 16)]
                        for rr in range(16):
                            wb = wvec[rr]
                            r = g * 16 + rr
                            for f in range(_F // 16):
                                sl = pl.ds(f * 16, 16)
                                rows[r, sl] = rows[r, sl] * wb
                        return rc

                    lax.fori_loop(0, _CHUNK // 16, per_group, 0)

                pltpu.sync_copy(rows, accum.at[dstv.at[j]], add=True)
                return cc

            lax.fori_loop(0, _NCH, per_chunk, 0)

            def adv_body(j, cc):
                for k in range(_CHUNK // 16):
                    sl = pl.ds(k * 16, 16)
                    srcv[j, sl] = srcv[j, sl] + _N
                return cc

            lax.fori_loop(0, _NCH, adv_body, 0)
            plsc.subcore_barrier()
            pltpu.sync_copy(accum.at[pl.ds(s * _RPT, _RPT)],
                            out_hbm.at[c, t, pl.ds(s * _RPT, _RPT)])
            return carry

        lax.fori_loop(0, _T, per_t, 0)

    return body(xT, ztile, srcv_h, dst16, w_sec)


_BN = 1000


def _tc_body(S_ref, Wz_ref, bz_ref, Wr_ref, br_ref, Wh_ref, bh_ref,
             Lz_ref, blz_ref, Lr_ref, blr_ref, Lh_ref, blh_ref,
             att_ref, W1_ref, b1_ref, W2_ref, b2_ref, y_ref, hacc_ref):
    f32 = jnp.float32
    Lzv, Lrv, Lhv = Lz_ref[...], Lr_ref[...], Lh_ref[...]
    Lzt, Lzb = Lzv[:256], Lzv[256:]
    Lrt, Lrb = Lrv[:256], Lrv[256:]
    Lht, Lhb = Lhv[:256], Lhv[256:]
    M8 = jnp.concatenate([
        jnp.dot(Wz_ref[...], Lzt, preferred_element_type=f32),
        jnp.dot(Wr_ref[...], Lrt, preferred_element_type=f32),
        jnp.dot(Wh_ref[...], Lht, preferred_element_type=f32)], axis=1) * 0.125
    cvec = jnp.concatenate([
        jnp.dot(bz_ref[...], Lzt, preferred_element_type=f32) + blz_ref[...],
        jnp.dot(br_ref[...], Lrt, preferred_element_type=f32) + blr_ref[...],
        jnp.dot(bh_ref[...], Lht, preferred_element_type=f32) + blh_ref[...]],
        axis=1)
    Lzr = jnp.concatenate([Lzb, Lrb], axis=1)
    probs = jax.nn.softmax(att_ref[...], axis=-1)

    def step(t, carry):
        H, Hacc = carry
        A = S_ref[pl.ds(t, 1)][0] + S_ref[pl.ds(t + _T, 1)][0]
        P = jnp.dot(A, M8, preferred_element_type=f32) + cvec
        ZR = P[:, :512] + jnp.dot(H, Lzr, preferred_element_type=f32)
        Z = jax.nn.sigmoid(ZR[:, :256])
        R = jax.nn.sigmoid(ZR[:, 256:])
        Htil = jnp.tanh(P[:, 512:]
                        + jnp.dot(H * R, Lhb, preferred_element_type=f32))
        Hn = Z * H + (1.0 - Z) * Htil
        mask = lax.broadcasted_iota(jnp.int32, (1, _T), 1) == t
        pt = jnp.sum(jnp.where(mask, probs, 0.0))
        return Hn, Hacc + pt * Hn

    H0 = jnp.zeros((_BN, 256), f32)
    _, Hacc = lax.fori_loop(0, _T, step, (H0, H0))
    h = jnp.maximum(Hacc, 0.0)
    h = jnp.maximum(
        jnp.dot(h, W1_ref[...], preferred_element_type=f32) + b1_ref[...], 0.0)
    y = jnp.dot(h, W2_ref[...], preferred_element_type=f32) + b2_ref[...]
    y_ref[...] = y
    hacc_ref[...] = Hacc


def _tc_gru(S2, Wz, bz, Wr, br, Wh, bh, Lz, blz, Lr, blr, Lh, blh,
            att, W1, b1, W2, b2):
    full = lambda shape: pl.BlockSpec(shape, lambda i: (0,) * len(shape))
    return pl.pallas_call(
        _tc_body,
        grid=(_N // _BN,),
        in_specs=[
            pl.BlockSpec((_NC * _T, _BN, _F), lambda i: (0, i, 0)),
            full((_F, 256)), full((1, 256)),
            full((_F, 256)), full((1, 256)),
            full((_F, 256)), full((1, 256)),
            full((512, 256)), full((1, 256)),
            full((512, 256)), full((1, 256)),
            full((512, 256)), full((1, 256)),
            full((1, _T)),
            full((256, 128)), full((1, 128)),
            full((128, 1)), full((1, 1)),
        ],
        out_specs=[
            pl.BlockSpec((_BN, 1), lambda i: (i, 0)),
            pl.BlockSpec((_BN, 256), lambda i: (i, 0)),
        ],
        out_shape=[
            jax.ShapeDtypeStruct((_N, 1), jnp.float32),
            jax.ShapeDtypeStruct((_N, 256), jnp.float32),
        ],
        compiler_params=pltpu.CompilerParams(
            dimension_semantics=("arbitrary",)),
    )(S2, Wz, bz, Wr, br, Wh, bh, Lz, blz, Lr, blr, Lh, blh,
      att, W1, b1, W2, b2)


def kernel(x, edge_index, IAedge_index, KSedge_index, KYedge_index,
           OHedge_index, WIedge_index, IAedge_attr, KSedge_attr, KYedge_attr,
           OHedge_attr, WIedge_attr, Wz, bz, Wr, br, Wh, bh, Lz, blz, Lr, blr,
           Lh, blh, att, W1, b1, W2, b2):
    xT = jnp.transpose(x, (2, 0, 1)).reshape(_T * _N, _F)
    xT = jnp.concatenate(
        [xT, jnp.zeros((_NP - _N, _F), jnp.float32)], axis=0)
    regs = [(IAedge_index, IAedge_attr), (KSedge_index, KSedge_attr),
            (KYedge_index, KYedge_attr), (OHedge_index, OHedge_attr),
            (WIedge_index, WIedge_attr)]
    srcv_h, dst16, w_sec = _prep_edges(edge_index, regs)
    ztile = jnp.zeros((_RPT, _F), jnp.float32)
    S = _sc_segsum(xT, ztile, srcv_h, dst16, w_sec)
    S2 = S.reshape(_NC * _T, _NP, _F)
    r2 = lambda v: v.reshape(1, -1)
    y, hacc = _tc_gru(S2, Wz, r2(bz), Wr, r2(br), Wh, r2(bh),
                      Lz, r2(blz), Lr, r2(blr), Lh, r2(blh),
                      r2(att), W1, r2(b1), W2, r2(b2))
    return (y, hacc)
